# Initial kernel scaffold; baseline (speedup 1.0000x reference)
import jax
import jax.numpy as jnp
from jax.experimental import pallas as pl


def kernel(activated, expert_indices, expert_weights, mlp2_weight, mlp2_bias, residual_x):
    raise NotImplementedError("write your pallas kernel here")


# speedup vs baseline: 1.4702x; 1.4702x over previous
import jax
import jax.numpy as jnp
from jax.experimental import pallas as pl


def _moe_step(idx_ref, ew_ref, act_ref, w_ref, bias_ref, resid_ref, out_ref):
    e = pl.program_id(0)
    idx = idx_ref[...]
    ew = ew_ref[...]
    g = jnp.where(idx == e, ew, 0.0)
    g0 = g[:, 0:1]
    g1 = g[:, 1:2]
    act = act_ref[...]
    pe = g0 * act[:, :64] + g1 * act[:, 64:]
    w = w_ref[0]
    contrib = jax.lax.dot_general(
        pe, w, (((1,), (1,)), ((), ())), preferred_element_type=jnp.float32
    )
    contrib = contrib + (g0 + g1) * bias_ref[0]

    @pl.when(e == 0)
    def _init():
        out_ref[...] = resid_ref[...] + contrib

    @pl.when(e != 0)
    def _acc():
        out_ref[...] += contrib


def kernel(activated, expert_indices, expert_weights, mlp2_weight, mlp2_bias, residual_x):
    B, TOPK, D_FF = activated.shape
    E, D_MODEL, _ = mlp2_weight.shape
    idx = jnp.asarray(expert_indices, jnp.int32)
    act2d = activated.reshape(B, TOPK * D_FF)
    bias3d = mlp2_bias.reshape(E, 1, D_MODEL)

    return pl.pallas_call(
        _moe_step,
        grid=(E,),
        in_specs=[
            pl.BlockSpec((B, TOPK), lambda e: (0, 0)),
            pl.BlockSpec((B, TOPK), lambda e: (0, 0)),
            pl.BlockSpec((B, TOPK * D_FF), lambda e: (0, 0)),
            pl.BlockSpec((1, D_MODEL, D_FF), lambda e: (e, 0, 0)),
            pl.BlockSpec((1, 1, D_MODEL), lambda e: (e, 0, 0)),
            pl.BlockSpec((B, D_MODEL), lambda e: (0, 0)),
        ],
        out_specs=pl.BlockSpec((B, D_MODEL), lambda e: (0, 0)),
        out_shape=jax.ShapeDtypeStruct((B, D_MODEL), jnp.float32),
    )(idx, expert_weights, act2d, mlp2_weight, bias3d, residual_x)
